# baseline (speedup 1.0000x reference)
import functools

import jax
import jax.numpy as jnp
from jax import lax
from jax.experimental import pallas as pl
from jax.experimental.pallas import tpu as pltpu
from jax.experimental.pallas import tpu_sc as plsc

B = 16
L = 2048
V = 6561
VP = 6656
D = 768

NC = 2
NS = 16
NW = NC * NS
D_PER_W = D // NW
DC = 8
N_CHUNK = D_PER_W // DC
L16 = L // 16

RB = 128


def _transpose_body(cb_ref, out_ref):
    out_ref[...] = cb_ref[...].T


def _gather_body(codes_hbm, cbt_hbm, out_hbm, colbuf, codes_v, outbuf, sem):
    c = lax.axis_index("c")
    s = lax.axis_index("s")
    wid = s * NC + c

    for chunk in range(N_CHUNK):
        d0 = wid * D_PER_W + chunk * DC
        pltpu.sync_copy(cbt_hbm.at[pl.ds(d0, DC)], colbuf)
        for b in range(B):
            pltpu.sync_copy(codes_hbm.at[b], codes_v)

            def l_body(i, _):
                idx = codes_v[pl.ds(i * 16, 16)]
                for r in range(DC):
                    row = jnp.full((16,), r, jnp.int32)
                    outbuf[r, pl.ds(i * 16, 16)] = plsc.load_gather(
                        colbuf, [row, idx])
                return 0

            lax.fori_loop(0, L16, l_body, 0)
            pltpu.sync_copy(outbuf, out_hbm.at[b, pl.ds(d0, DC), :])


@jax.jit
def _features(codes, codebook):
    cb_pad = jnp.pad(codebook, ((0, VP - V), (0, 0)))
    cbt = pl.pallas_call(
        _transpose_body,
        grid=(VP // RB,),
        in_specs=[pl.BlockSpec((RB, D), lambda i: (i, 0))],
        out_specs=pl.BlockSpec((D, RB), lambda i: (0, i)),
        out_shape=jax.ShapeDtypeStruct((D, VP), jnp.float32),
    )(cb_pad)

    mesh = plsc.VectorSubcoreMesh(core_axis_name="c", subcore_axis_name="s")
    f = functools.partial(
        pl.kernel,
        out_type=jax.ShapeDtypeStruct((B, D, L), jnp.float32),
        mesh=mesh,
        scratch_types=[
            pltpu.VMEM((DC, VP), jnp.float32),
            pltpu.VMEM((L,), jnp.int32),
            pltpu.VMEM((DC, L), jnp.float32),
            pltpu.SemaphoreType.DMA,
        ],
        compiler_params=pltpu.CompilerParams(
            use_tc_tiling_on_sc=False, needs_layout_passes=False),
    )(_gather_body)
    return f(codes, cbt)


def kernel(codes, codebook):
    return _features(codes.astype(jnp.int32), codebook)


# speedup vs baseline: 1.8935x; 1.8935x over previous
import functools

import jax
import jax.numpy as jnp
from jax import lax
from jax.experimental import pallas as pl
from jax.experimental.pallas import tpu as pltpu
from jax.experimental.pallas import tpu_sc as plsc

B = 16
L = 2048
V = 6561
VP = 6656
D = 768

NC = 2
NS = 16
NW = NC * NS
D_PER_W = D // NW
DC = 8
N_CHUNK = D_PER_W // DC

RB = 128


def _transpose_body(cb_ref, out_ref):
    out_ref[...] = cb_ref[...].T


def _gather_body(codes_hbm, cbt_hbm, out_hbm, colbuf, codes_v, outbuf, sem):
    c = lax.axis_index("c")
    s = lax.axis_index("s")
    wid = s * NC + c

    pltpu.sync_copy(codes_hbm, codes_v)
    splats = [jnp.full((16,), r, jnp.int32) for r in range(DC)]

    pending = {}
    for chunk in range(N_CHUNK):
        d0 = wid * D_PER_W + chunk * DC
        pltpu.sync_copy(cbt_hbm.at[pl.ds(d0, DC)], colbuf)
        for b in range(B):
            buf = (chunk * B + b) % 2
            if buf in pending:
                pending.pop(buf).wait()

            @plsc.parallel_loop(0, L, step=16, unroll=4)
            def _l_body(i):
                idx = codes_v[b, pl.ds(i, 16)]
                for r in range(DC):
                    outbuf[buf, r, pl.ds(i, 16)] = plsc.load_gather(
                        colbuf, [splats[r], idx])

            pending[buf] = pltpu.async_copy(
                outbuf.at[buf], out_hbm.at[b, pl.ds(d0, DC), :], sem)
    for d in pending.values():
        d.wait()


@jax.jit
def _features(codes, codebook):
    cb_pad = jnp.pad(codebook, ((0, VP - V), (0, 0)))
    cbt = pl.pallas_call(
        _transpose_body,
        grid=(VP // RB,),
        in_specs=[pl.BlockSpec((RB, D), lambda i: (i, 0))],
        out_specs=pl.BlockSpec((D, RB), lambda i: (0, i)),
        out_shape=jax.ShapeDtypeStruct((D, VP), jnp.float32),
    )(cb_pad)

    mesh = plsc.VectorSubcoreMesh(core_axis_name="c", subcore_axis_name="s")
    f = functools.partial(
        pl.kernel,
        out_type=jax.ShapeDtypeStruct((B, D, L), jnp.float32),
        mesh=mesh,
        scratch_types=[
            pltpu.VMEM((DC, VP), jnp.float32),
            pltpu.VMEM((B, L), jnp.int32),
            pltpu.VMEM((2, DC, L), jnp.float32),
            pltpu.SemaphoreType.DMA,
        ],
        compiler_params=pltpu.CompilerParams(
            use_tc_tiling_on_sc=False, needs_layout_passes=False),
    )(_gather_body)
    return f(codes, cbt)


def kernel(codes, codebook):
    return _features(codes.astype(jnp.int32), codebook)
